# baseline (speedup 1.0000x reference)
import functools

import jax
import jax.numpy as jnp
from jax import lax
from jax.experimental import pallas as pl
from jax.experimental.pallas import tpu as pltpu
from jax.experimental.pallas import tpu_sc as plsc

NUM_VTX = 227
NUM_DIM = 3
EMB = 128
BATCH = 256
SEQ = 2048
SCALE = float(EMB) ** 0.5

NC = 2
NS = 16
NW = NC * NS
S_BLK = 128
B_BLK = BATCH // 2
NBUF = 2


def _body(vert_hbm, vtx_hbm, pos_hbm, dim_hbm, out_hbm,
          idx1, table1, dim_v, comb_v, out_v, isem, o0, o1):
    osem = [o0, o1]
    wid = lax.axis_index("s") * NC + lax.axis_index("c")
    j = lax.rem(wid, 16)
    h = wid // 16
    s0 = j * S_BLK
    b0 = h * B_BLK

    pltpu.sync_copy(vtx_hbm, table1)
    pltpu.sync_copy(dim_hbm, dim_v)
    pltpu.sync_copy(pos_hbm.at[pl.ds(s0, S_BLK)], comb_v)

    def idx_row(b, carry):
        pltpu.async_copy(
            vert_hbm.at[pl.ds((b0 + b) * SEQ + s0, S_BLK)],
            idx1.at[pl.ds(b * S_BLK, S_BLK)], isem)
        return carry
    lax.fori_loop(0, B_BLK, idx_row, 0)

    def idx_drain(b, carry):
        pltpu.make_async_copy(
            vert_hbm.at[pl.ds(b0 * SEQ + s0, S_BLK)],
            idx1.at[pl.ds(0, S_BLK)], isem).wait()
        return carry
    lax.fori_loop(0, B_BLK, idx_drain, 0)

    def comb_row(i, carry):
        r = lax.rem(s0 + i, NUM_DIM)
        w0 = jnp.full((16,), jnp.where(r == 0, SCALE, 0.0), jnp.float32)
        w1 = jnp.full((16,), jnp.where(r == 1, SCALE, 0.0), jnp.float32)
        w2 = jnp.full((16,), jnp.where(r == 2, SCALE, 0.0), jnp.float32)
        for g in range(EMB // 16):
            sl = pl.ds(g * 16, 16)
            comb_v[i, sl] = (comb_v[i, sl] * SCALE
                             + dim_v[0, sl] * w0
                             + dim_v[1, sl] * w1
                             + dim_v[2, sl] * w2)
        return carry
    lax.fori_loop(0, S_BLK, comb_row, 0)

    col = lax.iota(jnp.int32, 16)

    def start_out(b, k):
        pltpu.async_copy(
            out_v.at[k], out_hbm.at[b0 + b, pl.ds(s0, S_BLK)], osem[k])

    def wait_out(b, k):
        pltpu.make_async_copy(
            out_v.at[k],
            out_hbm.at[b0 + b, pl.ds(s0, S_BLK)], osem[k]).wait()

    def batch_pair(bp, carry):
        for k in range(NBUF):
            b = bp * NBUF + k

            @pl.when(b >= NBUF)
            def _():
                wait_out(b - NBUF, k)

            def seq_row(i, c2):
                vvec = plsc.load_gather(
                    idx1, [jnp.full((16,), b * S_BLK + i, jnp.int32)])
                base = vvec * EMB + col
                for g in range(EMB // 16):
                    sl = pl.ds(g * 16, 16)
                    row = plsc.load_gather(table1, [base + g * 16])
                    out_v[k, i, sl] = row + comb_v[i, sl]
                return c2
            lax.fori_loop(0, S_BLK, seq_row, 0)
            start_out(b, k)
        return carry
    lax.fori_loop(0, B_BLK // NBUF, batch_pair, 0)

    for k in range(NBUF):
        wait_out(B_BLK - NBUF + k, k)


@jax.jit
def kernel(vertices, vtx_table, pos_table, dim_table):
    vert1 = vertices.astype(jnp.int32).reshape(-1)
    vtx1 = (vtx_table * SCALE).reshape(-1)
    mesh = plsc.VectorSubcoreMesh(core_axis_name="c", subcore_axis_name="s")
    f = functools.partial(
        pl.kernel,
        mesh=mesh,
        out_type=jax.ShapeDtypeStruct((BATCH, SEQ, EMB), jnp.float32),
        scratch_types=[
            pltpu.VMEM((B_BLK * S_BLK,), jnp.int32),
            pltpu.VMEM((NUM_VTX * EMB,), jnp.float32),
            pltpu.VMEM((NUM_DIM, EMB), jnp.float32),
            pltpu.VMEM((S_BLK, EMB), jnp.float32),
            pltpu.VMEM((NBUF, S_BLK, EMB), jnp.float32),
        ] + [pltpu.SemaphoreType.DMA] * 3,
        compiler_params=pltpu.CompilerParams(needs_layout_passes=False),
    )(_body)
    return f(vert1, vtx1, pos_table, dim_table)


# speedup vs baseline: 3.9871x; 3.9871x over previous
import functools

import jax
import jax.numpy as jnp
from jax import lax
from jax.experimental import pallas as pl
from jax.experimental.pallas import tpu as pltpu
from jax.experimental.pallas import tpu_sc as plsc

NUM_VTX = 227
NUM_DIM = 3
EMB = 128
BATCH = 256
SEQ = 2048
SCALE = float(EMB) ** 0.5

NC = 2
NS = 16
NW = NC * NS
S_BLK = 128
B_BLK = BATCH // 2
NBUF = 2


def _body(vert_hbm, vtx_hbm, pos_hbm, dim_hbm, out_hbm,
          idx1, table1, dim_v, comb_v, out_v, isem, o0, o1):
    osem = [o0, o1]
    wid = lax.axis_index("s") * NC + lax.axis_index("c")
    j = lax.rem(wid, 16)
    h = wid // 16
    s0 = j * S_BLK
    b0 = h * B_BLK

    pltpu.sync_copy(vtx_hbm, table1)
    pltpu.sync_copy(dim_hbm, dim_v)
    pltpu.sync_copy(pos_hbm.at[pl.ds(s0, S_BLK)], comb_v)

    def idx_row(b, carry):
        pltpu.async_copy(
            vert_hbm.at[pl.ds((b0 + b) * SEQ + s0, S_BLK)],
            idx1.at[pl.ds(b * S_BLK, S_BLK)], isem)
        return carry
    lax.fori_loop(0, B_BLK, idx_row, 0)

    def idx_drain(b, carry):
        pltpu.make_async_copy(
            vert_hbm.at[pl.ds(b0 * SEQ + s0, S_BLK)],
            idx1.at[pl.ds(0, S_BLK)], isem).wait()
        return carry
    lax.fori_loop(0, B_BLK, idx_drain, 0)

    def comb_row(i, carry):
        r = lax.rem(s0 + i, NUM_DIM)
        w0 = jnp.full((16,), jnp.where(r == 0, SCALE, 0.0), jnp.float32)
        w1 = jnp.full((16,), jnp.where(r == 1, SCALE, 0.0), jnp.float32)
        w2 = jnp.full((16,), jnp.where(r == 2, SCALE, 0.0), jnp.float32)
        for g in range(EMB // 16):
            sl = pl.ds(g * 16, 16)
            comb_v[i, sl] = (comb_v[i, sl] * SCALE
                             + dim_v[0, sl] * w0
                             + dim_v[1, sl] * w1
                             + dim_v[2, sl] * w2)
        return carry
    lax.fori_loop(0, S_BLK, comb_row, 0)

    col = lax.iota(jnp.int32, 16)

    def start_out(b, k):
        pltpu.async_copy(
            out_v.at[k], out_hbm.at[b0 + b, pl.ds(s0, S_BLK)], osem[k])

    def wait_out(b, k):
        pltpu.make_async_copy(
            out_v.at[k],
            out_hbm.at[b0 + b, pl.ds(s0, S_BLK)], osem[k]).wait()

    def batch_pair(bp, carry):
        for k in range(NBUF):
            b = bp * NBUF + k

            @pl.when(b >= NBUF)
            def _():
                wait_out(b - NBUF, k)

            @plsc.parallel_loop(0, S_BLK, step=1, unroll=8)
            def seq_row(i):
                vvec = plsc.load_gather(
                    idx1, [jnp.full((16,), b * S_BLK + i, jnp.int32)])
                base = vvec * EMB + col
                for g in range(EMB // 16):
                    sl = pl.ds(g * 16, 16)
                    row = plsc.load_gather(table1, [base + g * 16])
                    out_v[k, i, sl] = row + comb_v[i, sl]
            start_out(b, k)
        return carry
    lax.fori_loop(0, B_BLK // NBUF, batch_pair, 0)

    for k in range(NBUF):
        wait_out(B_BLK - NBUF + k, k)


@jax.jit
def kernel(vertices, vtx_table, pos_table, dim_table):
    vert1 = vertices.astype(jnp.int32).reshape(-1)
    vtx1 = (vtx_table * SCALE).reshape(-1)
    mesh = plsc.VectorSubcoreMesh(core_axis_name="c", subcore_axis_name="s")
    f = functools.partial(
        pl.kernel,
        mesh=mesh,
        out_type=jax.ShapeDtypeStruct((BATCH, SEQ, EMB), jnp.float32),
        scratch_types=[
            pltpu.VMEM((B_BLK * S_BLK,), jnp.int32),
            pltpu.VMEM((NUM_VTX * EMB,), jnp.float32),
            pltpu.VMEM((NUM_DIM, EMB), jnp.float32),
            pltpu.VMEM((S_BLK, EMB), jnp.float32),
            pltpu.VMEM((NBUF, S_BLK, EMB), jnp.float32),
        ] + [pltpu.SemaphoreType.DMA] * 3,
        compiler_params=pltpu.CompilerParams(needs_layout_passes=False),
    )(_body)
    return f(vert1, vtx1, pos_table, dim_table)
